# baseline (speedup 1.0000x reference)
import functools
import jax
import jax.numpy as jnp
from jax import lax
from jax.experimental import pallas as pl
from jax.experimental.pallas import tpu as pltpu
from jax.experimental.pallas import tpu_sc as plsc

_CH = 128


def _route_body(k_ref, a_ref, sidx_ref, zo_ref, z_ref, *, d, n_t):
    t = pl.program_id(0)
    g = pl.program_id(1)
    k = k_ref[:, pl.ds(g * d, d)]
    a = a_ref[g]
    TB = k.shape[0]
    M = a.shape[0]

    scores = lax.dot_general(k, a, (((1,), (1,)), ((), ())),
                             preferred_element_type=jnp.float32)
    mx = jnp.max(scores, axis=1, keepdims=True)
    negidx = lax.broadcasted_iota(jnp.int32, (TB, M), 1).astype(jnp.float32) * -1.0
    cand = jnp.where(scores == mx, negidx, -jnp.inf)
    topneg = jnp.max(cand, axis=1, keepdims=True)
    local = (g % 16) * M
    sidx_ref[0] = (local - topneg).astype(jnp.int32)

    onehot = (negidx == topneg).astype(jnp.float32)
    z = jnp.sum(onehot, axis=0)

    @pl.when(t == 0)
    def _init():
        z_ref[g, :] = z

    @pl.when(t > 0)
    def _acc():
        z_ref[g, :] += z

    zo_ref[0, 0] = z_ref[g, :]


def _scatter_body(k4, v4, gidx_h, sidx_h, zb_h,
                  ko, vo,
                  acc_sh, data, gidx, sidx, zb, sem):
    c = lax.axis_index("c")
    s = lax.axis_index("s")
    g = c * 16 + s
    n_chunk = gidx_h.shape[0] // (32 * _CH)
    rows = 512

    pltpu.sync_copy(zb_h, zb)
    for r in range(rows // _CH):
        pltpu.sync_copy(zb, acc_sh.at[pl.ds(s * rows + r * _CH, _CH)])
    plsc.subcore_barrier()

    def k_chunk(i, carry):
        base = g * (n_chunk * _CH) + i * _CH
        pltpu.sync_copy(gidx_h.at[pl.ds(base, _CH)], gidx)
        pltpu.sync_copy(sidx_h.at[pl.ds(base, _CH)], sidx)
        pltpu.async_copy(k4.at[gidx], data, sem).wait()
        pltpu.sync_copy(data, acc_sh.at[sidx], add=True)
        return carry

    lax.fori_loop(0, n_chunk, k_chunk, 0)

    for r in range(rows // _CH):
        pltpu.sync_copy(acc_sh.at[pl.ds(s * rows + r * _CH, _CH)],
                        ko.at[pl.ds(g * rows + r * _CH, _CH)])
    plsc.subcore_barrier()
    for r in range(rows // _CH):
        pltpu.sync_copy(zb, acc_sh.at[pl.ds(s * rows + r * _CH, _CH)])
    plsc.subcore_barrier()

    def v_chunk(i, carry):
        base = g * (n_chunk * _CH) + i * _CH
        pltpu.sync_copy(gidx_h.at[pl.ds(base, _CH)], gidx)
        pltpu.sync_copy(sidx_h.at[pl.ds(base, _CH)], sidx)
        pltpu.async_copy(v4.at[gidx], data, sem).wait()
        pltpu.sync_copy(data, acc_sh.at[sidx], add=True)
        return carry

    lax.fori_loop(0, n_chunk, v_chunk, 0)

    for r in range(rows // _CH):
        pltpu.sync_copy(acc_sh.at[pl.ds(s * rows + r * _CH, _CH)],
                        vo.at[pl.ds(g * rows + r * _CH, _CH)])


def _norm_body(ka_ref, va_ref, cnt_ref, ko_ref, vo_ref):
    zs = jnp.clip(cnt_ref[...], 1e-8, None)
    inv = 1.0 / zs
    ko_ref[...] = ka_ref[...] * inv
    vo_ref[...] = va_ref[...] * inv


def kernel(K_cold, V_cold, anchors):
    T, H, D = K_cold.shape
    G, M, _ = anchors.shape
    TB = 1024
    n_t = T // TB

    Kf = K_cold.reshape(T, H * D)

    sidx3, zcnt = pl.pallas_call(
        functools.partial(_route_body, d=D, n_t=n_t),
        grid=(n_t, G),
        in_specs=[
            pl.BlockSpec((TB, H * D), lambda t, g: (t, 0)),
            pl.BlockSpec((G, M, D), lambda t, g: (0, 0, 0)),
        ],
        out_specs=[
            pl.BlockSpec((1, TB, 1), lambda t, g: (g, t, 0)),
            pl.BlockSpec((1, 1, M), lambda t, g: (g, 0, 0)),
        ],
        out_shape=[
            jax.ShapeDtypeStruct((G, T, 1), jnp.int32),
            jax.ShapeDtypeStruct((G, 1, M), jnp.float32),
        ],
        scratch_shapes=[pltpu.VMEM((G, M), jnp.float32)],
    )(Kf, anchors)

    sidx_flat = sidx3.reshape(G * T)

    gidx_flat = (jnp.arange(T, dtype=jnp.int32)[None, :] * H
                 + jnp.arange(G, dtype=jnp.int32)[:, None]).reshape(G * T)

    K4 = K_cold.reshape(T * H, D)
    V4 = V_cold.reshape(T * H, D)
    zb = jnp.zeros((_CH, D), jnp.float32)

    mesh = plsc.VectorSubcoreMesh(core_axis_name="c", subcore_axis_name="s")
    scatter = functools.partial(
        pl.kernel,
        mesh=mesh,
        out_type=[
            jax.ShapeDtypeStruct((G * M, D), jnp.float32),
            jax.ShapeDtypeStruct((G * M, D), jnp.float32),
        ],
        scratch_types=[
            pltpu.VMEM_SHARED((16 * M, D), jnp.float32),
            pltpu.VMEM((_CH, D), jnp.float32),
            pltpu.VMEM((_CH,), jnp.int32),
            pltpu.VMEM((_CH,), jnp.int32),
            pltpu.VMEM((_CH, D), jnp.float32),
            pltpu.SemaphoreType.DMA,
        ],
    )(_scatter_body)
    k_acc, v_acc = scatter(K4, V4, gidx_flat, sidx_flat, zb)

    NB = 2048
    k_n, v_n = pl.pallas_call(
        _norm_body,
        grid=(G * M // NB,),
        in_specs=[
            pl.BlockSpec((NB, D), lambda i: (i, 0)),
            pl.BlockSpec((NB, D), lambda i: (i, 0)),
            pl.BlockSpec((NB, 1), lambda i: (i, 0)),
        ],
        out_specs=[
            pl.BlockSpec((NB, D), lambda i: (i, 0)),
            pl.BlockSpec((NB, D), lambda i: (i, 0)),
        ],
        out_shape=[
            jax.ShapeDtypeStruct((G * M, D), jnp.float32),
            jax.ShapeDtypeStruct((G * M, D), jnp.float32),
        ],
    )(k_acc, v_acc, zcnt.reshape(G * M, 1))

    K_mem = jnp.transpose(k_n.reshape(G, M, D), (1, 0, 2)).astype(K_cold.dtype)
    V_mem = jnp.transpose(v_n.reshape(G, M, D), (1, 0, 2)).astype(V_cold.dtype)
    return (K_mem, V_mem)


# speedup vs baseline: 1.1258x; 1.1258x over previous
import functools
import jax
import jax.numpy as jnp
from jax import lax
from jax.experimental import pallas as pl
from jax.experimental.pallas import tpu as pltpu
from jax.experimental.pallas import tpu_sc as plsc

_CH = 128


def _route_body(k_ref, a_ref, sidx_ref, zo_ref, z_ref, *, d, n_t):
    t = pl.program_id(0)
    g = pl.program_id(1)
    k = k_ref[:, pl.ds(g * d, d)]
    a = a_ref[g]
    TB = k.shape[0]
    M = a.shape[0]

    scores = lax.dot_general(k, a, (((1,), (1,)), ((), ())),
                             preferred_element_type=jnp.float32)
    mx = jnp.max(scores, axis=1, keepdims=True)
    negidx = lax.broadcasted_iota(jnp.int32, (TB, M), 1).astype(jnp.float32) * -1.0
    cand = jnp.where(scores == mx, negidx, -jnp.inf)
    topneg = jnp.max(cand, axis=1, keepdims=True)
    local = (g % 16) * M
    top = (local - topneg).astype(jnp.int32)
    sidx_ref[0] = top.reshape(TB // 128, 128)

    onehot = (negidx == topneg).astype(jnp.float32)
    z = jnp.sum(onehot, axis=0)

    @pl.when(t == 0)
    def _init():
        z_ref[g, :] = z

    @pl.when(t > 0)
    def _acc():
        z_ref[g, :] += z

    zo_ref[0, 0] = z_ref[g, :]


def _scatter_body(k4, v4, sidx_h, zb_h,
                  ko, vo,
                  acc_sh, data, gidx, sidx, zb, sem):
    c = lax.axis_index("c")
    s = lax.axis_index("s")
    g = c * 16 + s
    h = 32
    n_chunk = sidx_h.shape[0] // (32 * _CH)
    rows = 512

    pltpu.sync_copy(zb_h, zb)
    for r in range(rows // _CH):
        pltpu.sync_copy(zb, acc_sh.at[pl.ds(s * rows + r * _CH, _CH)])
    plsc.subcore_barrier()

    def fill_gidx(i):
        for j in range(_CH // 16):
            lane = lax.iota(jnp.int32, 16)
            gidx[pl.ds(j * 16, 16)] = (i * _CH + j * 16 + lane) * h + g

    def k_chunk(i, carry):
        base = g * (n_chunk * _CH) + i * _CH
        fill_gidx(i)
        pltpu.sync_copy(sidx_h.at[pl.ds(base, _CH)], sidx)
        pltpu.async_copy(k4.at[gidx], data, sem).wait()
        pltpu.sync_copy(data, acc_sh.at[sidx], add=True)
        return carry

    lax.fori_loop(0, n_chunk, k_chunk, 0)

    for r in range(rows // _CH):
        pltpu.sync_copy(acc_sh.at[pl.ds(s * rows + r * _CH, _CH)],
                        ko.at[pl.ds(g * rows + r * _CH, _CH)])
    plsc.subcore_barrier()
    for r in range(rows // _CH):
        pltpu.sync_copy(zb, acc_sh.at[pl.ds(s * rows + r * _CH, _CH)])
    plsc.subcore_barrier()

    def v_chunk(i, carry):
        base = g * (n_chunk * _CH) + i * _CH
        fill_gidx(i)
        pltpu.sync_copy(sidx_h.at[pl.ds(base, _CH)], sidx)
        pltpu.async_copy(v4.at[gidx], data, sem).wait()
        pltpu.sync_copy(data, acc_sh.at[sidx], add=True)
        return carry

    lax.fori_loop(0, n_chunk, v_chunk, 0)

    for r in range(rows // _CH):
        pltpu.sync_copy(acc_sh.at[pl.ds(s * rows + r * _CH, _CH)],
                        vo.at[pl.ds(g * rows + r * _CH, _CH)])


def _norm_body(ka_ref, va_ref, cnt_ref, ko_ref, vo_ref):
    zs = jnp.clip(cnt_ref[...], 1e-8, None)
    inv = 1.0 / zs
    ko_ref[...] = ka_ref[...] * inv
    vo_ref[...] = va_ref[...] * inv


def kernel(K_cold, V_cold, anchors):
    T, H, D = K_cold.shape
    G, M, _ = anchors.shape
    TB = 1024
    n_t = T // TB

    Kf = K_cold.reshape(T, H * D)

    sidx3, zcnt = pl.pallas_call(
        functools.partial(_route_body, d=D, n_t=n_t),
        grid=(n_t, G),
        in_specs=[
            pl.BlockSpec((TB, H * D), lambda t, g: (t, 0)),
            pl.BlockSpec((G, M, D), lambda t, g: (0, 0, 0)),
        ],
        out_specs=[
            pl.BlockSpec((1, TB // 128, 128), lambda t, g: (g, t, 0)),
            pl.BlockSpec((1, 1, M), lambda t, g: (g, 0, 0)),
        ],
        out_shape=[
            jax.ShapeDtypeStruct((G, T // 128, 128), jnp.int32),
            jax.ShapeDtypeStruct((G, 1, M), jnp.float32),
        ],
        scratch_shapes=[pltpu.VMEM((G, M), jnp.float32)],
    )(Kf, anchors)

    sidx_flat = sidx3.reshape(G * T)

    K4 = K_cold.reshape(T * H, D)
    V4 = V_cold.reshape(T * H, D)
    zb = jnp.zeros((_CH, D), jnp.float32)

    mesh = plsc.VectorSubcoreMesh(core_axis_name="c", subcore_axis_name="s")
    scatter = functools.partial(
        pl.kernel,
        mesh=mesh,
        out_type=[
            jax.ShapeDtypeStruct((G * M, D), jnp.float32),
            jax.ShapeDtypeStruct((G * M, D), jnp.float32),
        ],
        scratch_types=[
            pltpu.VMEM_SHARED((16 * M, D), jnp.float32),
            pltpu.VMEM((_CH, D), jnp.float32),
            pltpu.VMEM((_CH,), jnp.int32),
            pltpu.VMEM((_CH,), jnp.int32),
            pltpu.VMEM((_CH, D), jnp.float32),
            pltpu.SemaphoreType.DMA,
        ],
    )(_scatter_body)
    k_acc, v_acc = scatter(K4, V4, sidx_flat, zb)

    NB = 2048
    k_n, v_n = pl.pallas_call(
        _norm_body,
        grid=(G * M // NB,),
        in_specs=[
            pl.BlockSpec((NB, D), lambda i: (i, 0)),
            pl.BlockSpec((NB, D), lambda i: (i, 0)),
            pl.BlockSpec((NB, 1), lambda i: (i, 0)),
        ],
        out_specs=[
            pl.BlockSpec((NB, D), lambda i: (i, 0)),
            pl.BlockSpec((NB, D), lambda i: (i, 0)),
        ],
        out_shape=[
            jax.ShapeDtypeStruct((G * M, D), jnp.float32),
            jax.ShapeDtypeStruct((G * M, D), jnp.float32),
        ],
    )(k_acc, v_acc, zcnt.reshape(G * M, 1))

    K_mem = jnp.transpose(k_n.reshape(G, M, D), (1, 0, 2)).astype(K_cold.dtype)
    V_mem = jnp.transpose(v_n.reshape(G, M, D), (1, 0, 2)).astype(V_cold.dtype)
    return (K_mem, V_mem)


# speedup vs baseline: 1.9321x; 1.7162x over previous
import functools
import jax
import jax.numpy as jnp
from jax import lax
from jax.experimental import pallas as pl
from jax.experimental.pallas import tpu as pltpu


def _compactor_body(k_ref, v_ref, a_ref, ko_ref, vo_ref, z_ref, *, n_t):
    t = pl.program_id(1)
    k = k_ref[0]
    v = v_ref[0]
    a = a_ref[0]
    TB, D = k.shape
    M = a.shape[0]

    scores = lax.dot_general(k, a, (((1,), (1,)), ((), ())),
                             preferred_element_type=jnp.float32)
    mx = jnp.max(scores, axis=1, keepdims=True)
    negidx = lax.broadcasted_iota(jnp.int32, (TB, M), 1).astype(jnp.float32) * -1.0
    cand = jnp.where(scores == mx, negidx, -jnp.inf)
    topneg = jnp.max(cand, axis=1, keepdims=True)
    onehot = (negidx == topneg).astype(jnp.float32)

    oh16 = onehot.astype(jnp.bfloat16)
    ck = lax.dot_general(oh16, k.astype(jnp.bfloat16), (((0,), (0,)), ((), ())),
                         preferred_element_type=jnp.float32)
    cv = lax.dot_general(oh16, v.astype(jnp.bfloat16), (((0,), (0,)), ((), ())),
                         preferred_element_type=jnp.float32)
    z = jnp.sum(onehot, axis=0)[None, :]

    @pl.when(t == 0)
    def _init():
        ko_ref[0] = ck
        vo_ref[0] = cv
        z_ref[...] = z

    @pl.when(t > 0)
    def _acc():
        ko_ref[0] += ck
        vo_ref[0] += cv
        z_ref[...] += z

    @pl.when(t == n_t - 1)
    def _norm():
        zs = jnp.clip(z_ref[...], 1e-8, None)[0, :, None]
        ko_ref[0] = ko_ref[0] / zs
        vo_ref[0] = vo_ref[0] / zs


def kernel(K_cold, V_cold, anchors):
    T, H, D = K_cold.shape
    G, M, _ = anchors.shape
    TB = min(8192, T)
    n_t = T // TB

    Kg = jnp.transpose(K_cold, (1, 0, 2))
    Vg = jnp.transpose(V_cold, (1, 0, 2))

    grid = (G, n_t)
    out_shape = [
        jax.ShapeDtypeStruct((G, M, D), jnp.float32),
        jax.ShapeDtypeStruct((G, M, D), jnp.float32),
    ]
    k_acc, v_acc = pl.pallas_call(
        functools.partial(_compactor_body, n_t=n_t),
        grid=grid,
        in_specs=[
            pl.BlockSpec((1, TB, D), lambda g, t: (g, t, 0)),
            pl.BlockSpec((1, TB, D), lambda g, t: (g, t, 0)),
            pl.BlockSpec((1, M, D), lambda g, t: (g, 0, 0)),
        ],
        out_specs=[
            pl.BlockSpec((1, M, D), lambda g, t: (g, 0, 0)),
            pl.BlockSpec((1, M, D), lambda g, t: (g, 0, 0)),
        ],
        scratch_shapes=[pltpu.VMEM((1, M), jnp.float32)],
        out_shape=out_shape,
    )(Kg, Vg, anchors)

    K_mem = jnp.transpose(k_acc, (1, 0, 2)).astype(K_cold.dtype)
    V_mem = jnp.transpose(v_acc, (1, 0, 2)).astype(V_cold.dtype)
    return (K_mem, V_mem)
